# baseline (speedup 1.0000x reference)
import functools

import jax
import jax.numpy as jnp
from jax import lax
from jax.experimental import pallas as pl
from jax.experimental.pallas import tpu as pltpu


def _iou_tile(gt_ref, bt_ref, eps=1e-6):
    gx1 = gt_ref[:, 0:1]
    gy1 = gt_ref[:, 1:2]
    gx2 = gt_ref[:, 2:3]
    gy2 = gt_ref[:, 3:4]
    bx1 = bt_ref[0:1, :]
    by1 = bt_ref[1:2, :]
    bx2 = bt_ref[2:3, :]
    by2 = bt_ref[3:4, :]
    garea = (gx2 - gx1) * (gy2 - gy1)
    barea = (bx2 - bx1) * (by2 - by1)
    a = jnp.concatenate([garea, jnp.ones_like(garea)], axis=1)
    b = jnp.concatenate([jnp.ones_like(barea), barea], axis=0)
    sum_ab = jax.lax.dot_general(
        a, b, (((1,), (0,)), ((), ())),
        precision=jax.lax.Precision.HIGHEST,
        preferred_element_type=jnp.float32)
    w = jnp.clip(jnp.minimum(gx2, bx2) - jnp.maximum(gx1, bx1), 0.0)
    h = jnp.clip(jnp.minimum(gy2, by2) - jnp.maximum(gy1, by1), 0.0)
    inter = w * h
    union = sum_ab - inter
    return inter / jnp.maximum(union, eps)


def _assign_kernel(bt_ref, gt_ref, out_ref,
                   gtm, ssum, smax, cmax, carg, af_s, *, G, NT):
    p = pl.program_id(0)
    j = pl.program_id(1)
    gidx = lax.broadcasted_iota(jnp.int32, (G, 1), 0)

    @pl.when(p == 0)
    def _phase0():
        iou = _iou_tile(gt_ref, bt_ref)
        m = jnp.max(iou, axis=0, keepdims=True)
        am = jnp.min(jnp.where(iou == m, gidx, G), axis=0, keepdims=True)
        cmax[pl.ds(j, 1), :] = m
        carg[pl.ds(j, 1), :] = am

        mj = jnp.max(iou, axis=1, keepdims=True)
        nio1 = j * NT + 1 + lax.broadcasted_iota(jnp.int32, (1, NT), 1)
        v = jnp.where(iou == mj, nio1, 0)
        sj = jnp.sum(v, axis=1, keepdims=True)
        lj = jnp.max(v, axis=1, keepdims=True)

        @pl.when(j == 0)
        def _():
            gtm[:, :] = mj
            ssum[:, :] = sj
            smax[:, :] = lj

        @pl.when(j > 0)
        def _():
            run = gtm[:, :]
            better = mj > run
            equal = mj == run
            gtm[:, :] = jnp.maximum(run, mj)
            ssum[:, :] = jnp.where(better, sj,
                                   jnp.where(equal, ssum[:, :] + sj,
                                             ssum[:, :]))
            smax[:, :] = jnp.where(better, lj,
                                   jnp.where(equal,
                                             jnp.maximum(smax[:, :], lj),
                                             smax[:, :]))

    @pl.when((p == 1) & (j == 0))
    def _flagcheck():
        af_s[0] = jnp.max(jnp.where(ssum[:, :] != smax[:, :], 1, 0))

    @pl.when(p == 1)
    def _phase1():
        m = cmax[pl.ds(j, 1), :]
        am = carg[pl.ds(j, 1), :]
        assigned = jnp.where(m >= 0.5, am + 1, 0)
        nio1 = j * NT + 1 + lax.broadcasted_iota(jnp.int32, (1, NT), 1)
        colv = jnp.where(ssum[:, :] == smax[:, :], smax[:, :], -2)
        last = jnp.max(jnp.where(colv == nio1, gidx + 1, 0), axis=0,
                       keepdims=True)
        out_ref[:, :] = jnp.where(last > 0, last, assigned)

        @pl.when(af_s[0] > 0)
        def _exact_fallback():
            iou = _iou_tile(gt_ref, bt_ref)
            lastf = jnp.max(jnp.where(iou == gtm[:, :], gidx + 1, 0), axis=0,
                            keepdims=True)
            out_ref[:, :] = jnp.where(lastf > 0, lastf, assigned)


@jax.jit
def kernel(bboxes, gt_bboxes):
    N = bboxes.shape[0]
    G = gt_bboxes.shape[0]
    NT = 2560
    NP = ((N + NT - 1) // NT) * NT
    C = NP // NT
    bt = jnp.pad(bboxes, ((0, NP - N), (0, 0))).T

    out = pl.pallas_call(
        functools.partial(_assign_kernel, G=G, NT=NT),
        grid=(2, C),
        in_specs=[
            pl.BlockSpec((4, NT), lambda p, j: (0, j)),
            pl.BlockSpec((G, 4), lambda p, j: (0, 0)),
        ],
        out_specs=pl.BlockSpec((1, NT), lambda p, j: (0, j)),
        out_shape=jax.ShapeDtypeStruct((1, NP), jnp.int32),
        scratch_shapes=[
            pltpu.VMEM((G, 1), jnp.float32),
            pltpu.VMEM((G, 1), jnp.int32),
            pltpu.VMEM((G, 1), jnp.int32),
            pltpu.VMEM((C, NT), jnp.float32),
            pltpu.VMEM((C, NT), jnp.int32),
            pltpu.SMEM((1,), jnp.int32),
        ],
    )(bt, gt_bboxes)
    return out[0, :N]


# speedup vs baseline: 1.2054x; 1.2054x over previous
import functools

import jax
import jax.numpy as jnp
from jax import lax
from jax.experimental import pallas as pl
from jax.experimental.pallas import tpu as pltpu


def _iou_tile(gt_ref, bt_ref, eps=1e-6):
    gx1 = gt_ref[:, 0:1]
    gy1 = gt_ref[:, 1:2]
    gx2 = gt_ref[:, 2:3]
    gy2 = gt_ref[:, 3:4]
    bx1 = bt_ref[0:1, :]
    by1 = bt_ref[1:2, :]
    bx2 = bt_ref[2:3, :]
    by2 = bt_ref[3:4, :]
    garea = (gx2 - gx1) * (gy2 - gy1)
    barea = (bx2 - bx1) * (by2 - by1)
    w = jnp.clip(jnp.minimum(gx2, bx2) - jnp.maximum(gx1, bx1), 0.0)
    h = jnp.clip(jnp.minimum(gy2, by2) - jnp.maximum(gy1, by1), 0.0)
    inter = w * h
    union = garea + barea - inter
    return inter / jnp.maximum(union, eps)


def _assign_kernel(bt_ref, gt_ref, out_ref,
                   gtm, ssum, smax, cmax, carg, af_s, *, G, NT):
    p = pl.program_id(0)
    j = pl.program_id(1)
    gidx = lax.broadcasted_iota(jnp.int32, (G, 1), 0)

    @pl.when(p == 0)
    def _phase0():
        iou = _iou_tile(gt_ref, bt_ref)
        m = jnp.max(iou, axis=0, keepdims=True)
        am = jnp.min(jnp.where(iou == m, gidx, G), axis=0, keepdims=True)
        cmax[pl.ds(j, 1), :] = m
        carg[pl.ds(j, 1), :] = am

        mj = jnp.max(iou, axis=1, keepdims=True)
        nio1 = j * NT + 1 + lax.broadcasted_iota(jnp.int32, (1, NT), 1)
        v = jnp.where(iou == mj, nio1, 0)
        sj = jnp.sum(v, axis=1, keepdims=True)
        lj = jnp.max(v, axis=1, keepdims=True)

        @pl.when(j == 0)
        def _():
            gtm[:, :] = mj
            ssum[:, :] = sj
            smax[:, :] = lj

        @pl.when(j > 0)
        def _():
            run = gtm[:, :]
            better = mj > run
            equal = mj == run
            gtm[:, :] = jnp.maximum(run, mj)
            ssum[:, :] = jnp.where(better, sj,
                                   jnp.where(equal, ssum[:, :] + sj,
                                             ssum[:, :]))
            smax[:, :] = jnp.where(better, lj,
                                   jnp.where(equal,
                                             jnp.maximum(smax[:, :], lj),
                                             smax[:, :]))

    @pl.when((p == 1) & (j == 0))
    def _flagcheck():
        af_s[0] = jnp.max(jnp.where(ssum[:, :] != smax[:, :], 1, 0))

    @pl.when(p == 1)
    def _phase1():
        m = cmax[pl.ds(j, 1), :]
        am = carg[pl.ds(j, 1), :]
        assigned = jnp.where(m >= 0.5, am + 1, 0)
        nio1 = j * NT + 1 + lax.broadcasted_iota(jnp.int32, (1, NT), 1)
        colv = jnp.where(ssum[:, :] == smax[:, :], smax[:, :], -2)
        last = jnp.max(jnp.where(colv == nio1, gidx + 1, 0), axis=0,
                       keepdims=True)
        out_ref[:, :] = jnp.where(last > 0, last, assigned)

        @pl.when(af_s[0] > 0)
        def _exact_fallback():
            iou = _iou_tile(gt_ref, bt_ref)
            lastf = jnp.max(jnp.where(iou == gtm[:, :], gidx + 1, 0), axis=0,
                            keepdims=True)
            out_ref[:, :] = jnp.where(lastf > 0, lastf, assigned)


@jax.jit
def kernel(bboxes, gt_bboxes):
    N = bboxes.shape[0]
    G = gt_bboxes.shape[0]
    NT = 2560
    NP = ((N + NT - 1) // NT) * NT
    C = NP // NT
    bt = jnp.pad(bboxes, ((0, NP - N), (0, 0))).T

    out = pl.pallas_call(
        functools.partial(_assign_kernel, G=G, NT=NT),
        grid=(2, C),
        in_specs=[
            pl.BlockSpec((4, NT), lambda p, j: (0, j)),
            pl.BlockSpec((G, 4), lambda p, j: (0, 0)),
        ],
        out_specs=pl.BlockSpec((1, NT), lambda p, j: (0, j)),
        out_shape=jax.ShapeDtypeStruct((1, NP), jnp.int32),
        scratch_shapes=[
            pltpu.VMEM((G, 1), jnp.float32),
            pltpu.VMEM((G, 1), jnp.int32),
            pltpu.VMEM((G, 1), jnp.int32),
            pltpu.VMEM((C, NT), jnp.float32),
            pltpu.VMEM((C, NT), jnp.int32),
            pltpu.SMEM((1,), jnp.int32),
        ],
    )(bt, gt_bboxes)
    return out[0, :N]
